# baseline (speedup 1.0000x reference)
import jax
import jax.numpy as jnp
from jax import lax
from jax.experimental import pallas as pl
from jax.experimental.pallas import tpu as pltpu
from jax.experimental.pallas import tpu_sc as plsc

_CH = 32


def kernel(x, pe_weight):
    B, L = x.shape
    M, D = pe_weight.shape
    info = plsc.get_sparse_core_info()
    NC, NS = info.num_cores, info.num_subcores
    NW = NC * NS
    rows_w = M // NW
    n_chunks = rows_w // _CH

    mesh = plsc.VectorSubcoreMesh(core_axis_name="c", subcore_axis_name="s")

    def body(w_hbm, o_hbm, buf, in_sem, out_sems):
        wid = lax.axis_index("s") * NC + lax.axis_index("c")
        base = wid * rows_w
        for g in range(n_chunks):
            slot = g % 2
            if g >= 2:
                for b in range(B):
                    pltpu.make_async_copy(
                        buf.at[slot],
                        o_hbm.at[b, pl.ds(base + (g - 2) * _CH, _CH)],
                        out_sems.at[slot, b],
                    ).wait()
            pltpu.make_async_copy(
                w_hbm.at[pl.ds(base + g * _CH, _CH)],
                buf.at[slot],
                in_sem,
            ).start()
            pltpu.make_async_copy(
                w_hbm.at[pl.ds(base + g * _CH, _CH)],
                buf.at[slot],
                in_sem,
            ).wait()
            for b in range(B):
                pltpu.make_async_copy(
                    buf.at[slot],
                    o_hbm.at[b, pl.ds(base + g * _CH, _CH)],
                    out_sems.at[slot, b],
                ).start()
        for g in (n_chunks - 2, n_chunks - 1):
            slot = g % 2
            for b in range(B):
                pltpu.make_async_copy(
                    buf.at[slot],
                    o_hbm.at[b, pl.ds(base + g * _CH, _CH)],
                    out_sems.at[slot, b],
                ).wait()

    sc_fn = pl.kernel(
        body,
        mesh=mesh,
        out_type=jax.ShapeDtypeStruct((B, L, D), pe_weight.dtype),
        scratch_types=[
            pltpu.VMEM((2, _CH, D), pe_weight.dtype),
            pltpu.SemaphoreType.DMA,
            pltpu.SemaphoreType.DMA((2, B)),
        ],
    )
    return sc_fn(pe_weight)


# speedup vs baseline: 1.0669x; 1.0669x over previous
import jax
import jax.numpy as jnp
from jax import lax
from jax.experimental import pallas as pl
from jax.experimental.pallas import tpu as pltpu
from jax.experimental.pallas import tpu_sc as plsc

_CH = 32
_SLOTS = 3


def kernel(x, pe_weight):
    B, L = x.shape
    M, D = pe_weight.shape
    info = plsc.get_sparse_core_info()
    NC, NS = info.num_cores, info.num_subcores
    NW = NC * NS
    rows_w = M // NW
    n_chunks = rows_w // _CH

    mesh = plsc.VectorSubcoreMesh(core_axis_name="c", subcore_axis_name="s")

    def body(w_hbm, o_hbm, buf, in_sems, out_sems):
        wid = lax.axis_index("s") * NC + lax.axis_index("c")
        base = wid * rows_w

        def in_copy(g):
            return pltpu.make_async_copy(
                w_hbm.at[pl.ds(base + g * _CH, _CH)],
                buf.at[g % _SLOTS],
                in_sems.at[g % _SLOTS],
            )

        def out_copy(g, b):
            return pltpu.make_async_copy(
                buf.at[g % _SLOTS],
                o_hbm.at[b, pl.ds(base + g * _CH, _CH)],
                out_sems.at[g % _SLOTS, b],
            )

        in_copy(0).start()
        if n_chunks > 1:
            in_copy(1).start()
        for g in range(n_chunks):
            in_copy(g).wait()
            for b in range(B):
                out_copy(g, b).start()
            nxt = g + 2
            if nxt < n_chunks:
                if nxt - _SLOTS >= 0:
                    for b in range(B):
                        out_copy(nxt - _SLOTS, b).wait()
                in_copy(nxt).start()
        for g in range(max(0, n_chunks - _SLOTS), n_chunks):
            for b in range(B):
                out_copy(g, b).wait()

    sc_fn = pl.kernel(
        body,
        mesh=mesh,
        out_type=jax.ShapeDtypeStruct((B, L, D), pe_weight.dtype),
        scratch_types=[
            pltpu.VMEM((_SLOTS, _CH, D), pe_weight.dtype),
            pltpu.SemaphoreType.DMA((_SLOTS,)),
            pltpu.SemaphoreType.DMA((_SLOTS, B)),
        ],
    )
    return sc_fn(pe_weight)
